# Initial kernel scaffold; baseline (speedup 1.0000x reference)
import jax
import jax.numpy as jnp
from jax.experimental import pallas as pl


def kernel(nodes_m, edges_m, A_nodes, A_edges, Wn, bn, We, be, Wout, Win, Wu, bu, Wr, br, Wh, bh):
    raise NotImplementedError("write your pallas kernel here")


# speedup vs baseline: 103.5028x; 103.5028x over previous
import jax
import jax.numpy as jnp
from jax.experimental import pallas as pl
from jax.experimental.pallas import tpu as pltpu

B, N, EPN, D = 8, 512, 32, 256
STEPS = 5


def _gnn_body(nm_ref, em_ref, an_ref, ae_ref, Wn_ref, bn_ref, We_ref, be_ref,
              Wout_ref, Win_ref, Wu_ref, bu_ref, Wr_ref, br_ref, Wh_ref,
              bh_ref, out_ref):
    f32 = jnp.float32
    An = an_ref[0]
    Ae = ae_ref[0]

    iota_col = jax.lax.broadcasted_iota(jnp.int32, (N, N), 1)

    Mn = jnp.zeros((N, N), f32)
    Meo = jnp.zeros((N, N), f32)
    Mei = jnp.zeros((N, N), f32)
    for k in range(EPN):
        an_k = An[:, k][:, None]
        ae_k = Ae[:, k][:, None]
        Mn = Mn + (an_k == iota_col).astype(f32)
        Meo = Meo + (ae_k == iota_col).astype(f32)
        Mei = Mei + (ae_k == iota_col + N).astype(f32)
    col0 = (jax.lax.broadcasted_iota(jnp.int32, (1, N), 1) != 0).astype(f32)
    Mn = Mn * col0
    Meo = Meo * col0

    nodes_mask = (jnp.sum(An, axis=1) != 0).astype(f32)[:, None]
    edges_mask = (jnp.sum(Ae, axis=1) != 0).astype(f32)[:, None]

    def mm(a, b):
        return jnp.dot(a, b, preferred_element_type=f32)

    nm = nm_ref[0]
    em = em_ref[0]
    S = jnp.tanh(mm(nm, Wn_ref[...]) + bn_ref[...]) * nodes_mask
    row_iota = jax.lax.broadcasted_iota(jnp.int32, (N, 1), 0)
    S = jnp.where(row_iota == 1, 1.0, S)

    e = jnp.tanh(mm(em, We_ref[...]) + be_ref[...]) * edges_mask
    act_e = mm(Meo, mm(e, Wout_ref[...])) + mm(Mei, mm(e, Win_ref[...]))

    Wu_a, Wu_s = Wu_ref[:D, :], Wu_ref[D:, :]
    Wr_a, Wr_s = Wr_ref[:D, :], Wr_ref[D:, :]
    Wh_a, Wh_s = Wh_ref[:D, :], Wh_ref[D:, :]
    bu = bu_ref[...]
    br = br_ref[...]
    bh = bh_ref[...]

    for _ in range(STEPS):
        act = mm(Mn, S) + act_e
        u = jax.nn.sigmoid(mm(act, Wu_a) + mm(S, Wu_s) + bu)
        r = jax.nn.sigmoid(mm(act, Wr_a) + mm(S, Wr_s) + br)
        h = jnp.tanh(mm(act, Wh_a) + mm(r * S, Wh_s) + bh)
        S = S + u * (h - S)

    out_ref[...] = S[1, :][None, None, :]


def kernel(nodes_m, edges_m, A_nodes, A_edges, Wn, bn, We, be, Wout, Win,
           Wu, bu, Wr, br, Wh, bh):
    bn2, be2, bu2, br2, bh2 = (x.reshape(1, D) for x in (bn, be, bu, br, bh))
    grid = (B,)
    full2 = lambda shape: pl.BlockSpec(shape, lambda b: (0,) * len(shape))
    per_b3 = lambda d1, d2: pl.BlockSpec((1, d1, d2), lambda b: (b, 0, 0))
    return pl.pallas_call(
        _gnn_body,
        grid=grid,
        in_specs=[
            per_b3(N, D),
            per_b3(N, D),
            per_b3(N, EPN),
            per_b3(N, EPN),
            full2((D, D)),
            full2((1, D)),
            full2((D, D)),
            full2((1, D)),
            full2((D, D)),
            full2((D, D)),
            full2((2 * D, D)),
            full2((1, D)),
            full2((2 * D, D)),
            full2((1, D)),
            full2((2 * D, D)),
            full2((1, D)),
        ],
        out_specs=pl.BlockSpec((1, 1, D), lambda b: (b, 0, 0)),
        out_shape=jax.ShapeDtypeStruct((B, 1, D), jnp.float32),
        compiler_params=pltpu.CompilerParams(
            dimension_semantics=("arbitrary",)),
    )(nodes_m, edges_m, A_nodes, A_edges, Wn, bn2, We, be2, Wout, Win,
      Wu, bu2, Wr, br2, Wh, bh2).reshape(B, D)


# speedup vs baseline: 150.0827x; 1.4500x over previous
import functools

import jax
import jax.numpy as jnp
from jax import lax
from jax.experimental import pallas as pl
from jax.experimental.pallas import tpu as pltpu
from jax.experimental.pallas import tpu_sc as plsc

B, N, EPN, D = 8, 512, 32, 256
STEPS = 5

_NC, _NS = 2, 16
_NW = _NC * _NS
_RC = (B * N) // _NW
_L = 16


def _sc_build_body(anT_hbm, aeT_hbm, mn_hbm, meo_hbm, mei_hbm,
                   idxn_v, idxe_v, buf_v):
    wid = lax.axis_index("s") * _NC + lax.axis_index("c")
    base = wid * _RC
    pltpu.sync_copy(anT_hbm.at[:, pl.ds(base, _RC)], idxn_v)
    pltpu.sync_copy(aeT_hbm.at[:, pl.ds(base, _RC)], idxe_v)

    zero16 = jnp.zeros((_L,), jnp.float32)

    def zrow(i, carry):
        for j in range(N // _L):
            buf_v[i, pl.ds(j * _L, _L)] = zero16
        return carry

    lax.fori_loop(0, _RC, zrow, 0)

    lane = lax.iota(jnp.int32, _L)

    def scatter(idx_v, val, kind):
        def body(g, carry):
            row = lane + g * _L
            for k in range(EPN):
                idx = idx_v[k, pl.ds(g * _L, _L)]
                if kind == 0:
                    mask = (idx != 0) & (idx < N)
                    col = idx
                else:
                    mask = idx >= N
                    col = idx - N
                plsc.addupdate_scatter(buf_v, [row, col], val, mask=mask)
            return carry
        lax.fori_loop(0, _RC // _L, body, 0)

    ones = jnp.full((_L,), 1.0, jnp.float32)
    negs = jnp.full((_L,), -1.0, jnp.float32)

    scatter(idxn_v, ones, 0)
    pltpu.sync_copy(buf_v, mn_hbm.at[pl.ds(base, _RC)])
    scatter(idxn_v, negs, 0)

    scatter(idxe_v, ones, 0)
    pltpu.sync_copy(buf_v, meo_hbm.at[pl.ds(base, _RC)])
    scatter(idxe_v, negs, 0)

    scatter(idxe_v, ones, 1)
    pltpu.sync_copy(buf_v, mei_hbm.at[pl.ds(base, _RC)])


_sc_build = functools.partial(
    pl.kernel,
    out_type=(jax.ShapeDtypeStruct((B * N, N), jnp.float32),) * 3,
    mesh=plsc.VectorSubcoreMesh(core_axis_name="c", subcore_axis_name="s"),
    compiler_params=pltpu.CompilerParams(needs_layout_passes=False),
    scratch_types=[
        pltpu.VMEM((EPN, _RC), jnp.int32),
        pltpu.VMEM((EPN, _RC), jnp.int32),
        pltpu.VMEM((_RC, N), jnp.float32),
    ],
)(_sc_build_body)


def _gnn_body(nm_ref, em_ref, an_ref, ae_ref, mn_ref, meo_ref, mei_ref,
              Wn_ref, bn_ref, We_ref, be_ref, Wout_ref, Win_ref, Wu_ref,
              bu_ref, Wr_ref, br_ref, Wh_ref, bh_ref, out_ref):
    f32 = jnp.float32
    An = an_ref[0]
    Ae = ae_ref[0]
    Mn = mn_ref[...]
    Meo = meo_ref[...]
    Mei = mei_ref[...]

    nodes_mask = (jnp.sum(An, axis=1) != 0).astype(f32)[:, None]
    edges_mask = (jnp.sum(Ae, axis=1) != 0).astype(f32)[:, None]

    def mm(a, b):
        return jnp.dot(a, b, preferred_element_type=f32)

    nm = nm_ref[0]
    em = em_ref[0]
    S = jnp.tanh(mm(nm, Wn_ref[...]) + bn_ref[...]) * nodes_mask
    row_iota = jax.lax.broadcasted_iota(jnp.int32, (N, 1), 0)
    S = jnp.where(row_iota == 1, 1.0, S)

    e = jnp.tanh(mm(em, We_ref[...]) + be_ref[...]) * edges_mask
    act_e = mm(Meo, mm(e, Wout_ref[...])) + mm(Mei, mm(e, Win_ref[...]))

    Wu_a, Wu_s = Wu_ref[:D, :], Wu_ref[D:, :]
    Wr_a, Wr_s = Wr_ref[:D, :], Wr_ref[D:, :]
    Wh_a, Wh_s = Wh_ref[:D, :], Wh_ref[D:, :]
    bu = bu_ref[...]
    br = br_ref[...]
    bh = bh_ref[...]

    for _ in range(STEPS):
        act = mm(Mn, S) + act_e
        u = jax.nn.sigmoid(mm(act, Wu_a) + mm(S, Wu_s) + bu)
        r = jax.nn.sigmoid(mm(act, Wr_a) + mm(S, Wr_s) + br)
        h = jnp.tanh(mm(act, Wh_a) + mm(r * S, Wh_s) + bh)
        S = S + u * (h - S)

    out_ref[...] = S[1, :][None, None, :]


def kernel(nodes_m, edges_m, A_nodes, A_edges, Wn, bn, We, be, Wout, Win,
           Wu, bu, Wr, br, Wh, bh):
    anT = A_nodes.reshape(B * N, EPN).T
    aeT = A_edges.reshape(B * N, EPN).T
    Mn, Meo, Mei = _sc_build(anT, aeT)

    bn2, be2, bu2, br2, bh2 = (x.reshape(1, D) for x in (bn, be, bu, br, bh))
    full2 = lambda shape: pl.BlockSpec(shape, lambda b: (0,) * len(shape))
    per_b3 = lambda d1, d2: pl.BlockSpec((1, d1, d2), lambda b: (b, 0, 0))
    m_spec = pl.BlockSpec((N, N), lambda b: (b, 0))
    return pl.pallas_call(
        _gnn_body,
        grid=(B,),
        in_specs=[
            per_b3(N, D),
            per_b3(N, D),
            per_b3(N, EPN),
            per_b3(N, EPN),
            m_spec,
            m_spec,
            m_spec,
            full2((D, D)),
            full2((1, D)),
            full2((D, D)),
            full2((1, D)),
            full2((D, D)),
            full2((D, D)),
            full2((2 * D, D)),
            full2((1, D)),
            full2((2 * D, D)),
            full2((1, D)),
            full2((2 * D, D)),
            full2((1, D)),
        ],
        out_specs=pl.BlockSpec((1, 1, D), lambda b: (b, 0, 0)),
        out_shape=jax.ShapeDtypeStruct((B, 1, D), jnp.float32),
        compiler_params=pltpu.CompilerParams(
            dimension_semantics=("arbitrary",)),
    )(nodes_m, edges_m, A_nodes, A_edges, Mn, Meo, Mei, Wn, bn2, We, be2,
      Wout, Win, Wu, bu2, Wr, br2, Wh, bh2).reshape(B, D)
